# Initial kernel scaffold; baseline (speedup 1.0000x reference)
import jax
import jax.numpy as jnp
from jax.experimental import pallas as pl


def kernel(x, replace_vals, replace_idx):
    raise NotImplementedError("write your pallas kernel here")


# speedup vs baseline: 1.2678x; 1.2678x over previous
import functools

import jax
import jax.numpy as jnp
from jax.experimental import pallas as pl
from jax.experimental.pallas import tpu as pltpu

_BLK = 512


def _body(idx_ref, x_ref, vals_ref, out_ref):
    out_ref[...] = x_ref[...]
    blk_start = pl.program_id(0) * _BLK
    n_idx = idx_ref.shape[0]
    n_rep = vals_ref.shape[0]

    def step(k, carry):
        local = idx_ref[k] - blk_start

        @pl.when((local >= 0) & (local < _BLK))
        def _():
            v = k - (k // n_rep) * n_rep
            out_ref[pl.ds(local, 1), :] = vals_ref[pl.ds(v, 1), :]

        return carry

    jax.lax.fori_loop(0, n_idx, step, 0)


def kernel(x, replace_vals, replace_idx):
    b, s, d = x.shape
    n = replace_idx.shape[0]
    x2 = x.reshape(b * s, d)
    idx_all = (replace_idx[None, :] + (jnp.arange(b, dtype=jnp.int32) * s)[:, None]).reshape(-1)

    grid = (b * s) // _BLK
    out = pl.pallas_call(
        _body,
        grid_spec=pltpu.PrefetchScalarGridSpec(
            num_scalar_prefetch=1,
            grid=(grid,),
            in_specs=[
                pl.BlockSpec((_BLK, d), lambda i, idx: (i, 0)),
                pl.BlockSpec((n, d), lambda i, idx: (0, 0)),
            ],
            out_specs=pl.BlockSpec((_BLK, d), lambda i, idx: (i, 0)),
        ),
        out_shape=jax.ShapeDtypeStruct((b * s, d), x.dtype),
        compiler_params=pltpu.CompilerParams(
            dimension_semantics=("arbitrary",),
        ),
    )(idx_all, x2, replace_vals)
    return out.reshape(b, s, d)
